# Initial kernel scaffold; baseline (speedup 1.0000x reference)
import jax
import jax.numpy as jnp
from jax.experimental import pallas as pl


def kernel(features, src, dst, Wh, bh, alw, alb, arw, arb, Wo, bo, alow, alob, arow, arob):
    raise NotImplementedError("write your pallas kernel here")


# speedup vs baseline: 24.1562x; 24.1562x over previous
import functools

import jax
import jax.numpy as jnp
from jax import lax
from jax.experimental import pallas as pl
from jax.experimental.pallas import tpu as pltpu
from jax.experimental.pallas import tpu_sc as plsc

N = 10000
E = 320000
IN_DIM = 128
HID = 64
HEADS = 4
NCLS = 64

NPAD = 10240
NC, NS, L = 2, 16, 16
HID2 = HID // 2
CH = 80
NCHUNK = (E // NS) // CH
SEG = NPAD // NS
VPC = CH // L


def _dense1_body(x_ref, w_ref, b_ref, aw_ref, ab_ref, ft_ref, a12_ref):
    ft = jnp.dot(x_ref[...], w_ref[0], preferred_element_type=jnp.float32)
    ft = ft + b_ref[0]
    ft_ref[0, 0] = ft[:, :HID2]
    ft_ref[0, 1] = ft[:, HID2:]
    a12 = jnp.dot(ft, aw_ref[0], preferred_element_type=jnp.float32)
    a12_ref[0] = a12 + ab_ref[0]


def _dense1(xp, Wh, bh3, aw2, ab2):
    return pl.pallas_call(
        _dense1_body,
        grid=(HEADS,),
        in_specs=[
            pl.BlockSpec((NPAD, IN_DIM), lambda h: (0, 0)),
            pl.BlockSpec((1, IN_DIM, HID), lambda h: (h, 0, 0)),
            pl.BlockSpec((1, 1, HID), lambda h: (h, 0, 0)),
            pl.BlockSpec((1, HID, 2), lambda h: (h, 0, 0)),
            pl.BlockSpec((1, 1, 2), lambda h: (h, 0, 0)),
        ],
        out_specs=[
            pl.BlockSpec((1, NC, NPAD, HID2), lambda h: (h, 0, 0, 0)),
            pl.BlockSpec((1, NPAD, 2), lambda h: (h, 0, 0)),
        ],
        out_shape=[
            jax.ShapeDtypeStruct((HEADS, NC, NPAD, HID2), jnp.float32),
            jax.ShapeDtypeStruct((HEADS, NPAD, 2), jnp.float32),
        ],
    )(xp, Wh, bh3, aw2, ab2)


def _dense2_body(p0_ref, p1_ref, wa_ref, wb_ref, b_ref, aw_ref, ab_ref,
                 ft2_ref, fth_ref, a12_ref):
    h = pl.program_id(0)
    s0 = p0_ref[0]
    s1 = p1_ref[0]
    h0 = jnp.where(s0 > 0, s0, jnp.exp(s0) - 1.0)
    h1 = jnp.where(s1 > 0, s1, jnp.exp(s1) - 1.0)
    part = (jnp.dot(h0, wa_ref[0], preferred_element_type=jnp.float32)
            + jnp.dot(h1, wb_ref[0], preferred_element_type=jnp.float32))

    @pl.when(h == 0)
    def _():
        ft2_ref[...] = part + b_ref[...]

    @pl.when(h > 0)
    def _():
        ft2_ref[...] = ft2_ref[...] + part

    @pl.when(h == HEADS - 1)
    def _():
        ft2 = ft2_ref[...]
        fth_ref[0] = ft2[:, :HID2]
        fth_ref[1] = ft2[:, HID2:]
        a12_ref[...] = (jnp.dot(ft2, aw_ref[...],
                                preferred_element_type=jnp.float32)
                        + ab_ref[...])


def _dense2(p0, p1, Wo2a, Wo2b, bo2, aw22, ab22):
    return pl.pallas_call(
        _dense2_body,
        grid=(HEADS,),
        in_specs=[
            pl.BlockSpec((1, NPAD, HID2), lambda h: (h, 0, 0)),
            pl.BlockSpec((1, NPAD, HID2), lambda h: (h, 0, 0)),
            pl.BlockSpec((1, HID2, NCLS), lambda h: (h, 0, 0)),
            pl.BlockSpec((1, HID2, NCLS), lambda h: (h, 0, 0)),
            pl.BlockSpec((1, NCLS), lambda h: (0, 0)),
            pl.BlockSpec((NCLS, 2), lambda h: (0, 0)),
            pl.BlockSpec((1, 2), lambda h: (0, 0)),
        ],
        out_specs=[
            pl.BlockSpec((NPAD, NCLS), lambda h: (0, 0)),
            pl.BlockSpec((NC, NPAD, HID2), lambda h: (0, 0, 0)),
            pl.BlockSpec((NPAD, 2), lambda h: (0, 0)),
        ],
        out_shape=[
            jax.ShapeDtypeStruct((NPAD, NCLS), jnp.float32),
            jax.ShapeDtypeStruct((NC, NPAD, HID2), jnp.float32),
            jax.ShapeDtypeStruct((NPAD, 2), jnp.float32),
        ],
    )(p0, p1, Wo2a, Wo2b, bo2, aw22, ab22)


def _final_body(q0_ref, q1_ref, o_ref):
    s0 = q0_ref[...]
    s1 = q1_ref[...]
    o_ref[:, :HID2] = jnp.where(s0 > 0, s0, jnp.exp(s0) - 1.0)
    o_ref[:, HID2:] = jnp.where(s1 > 0, s1, jnp.exp(s1) - 1.0)


def _final(q0, q1):
    return pl.pallas_call(
        _final_body,
        out_shape=jax.ShapeDtypeStruct((NPAD, NCLS), jnp.float32),
    )(q0, q1)


def _sc_head(h, cid, sid, ft_hbm, a12_hbm, outp, slab_hbm,
             a12_v, amax_v, src_v, dst_v, a_v, rows_v, stage_v, sem):
    one16 = jnp.ones((L,), jnp.int32)
    fzero = jnp.zeros((L,), jnp.float32)

    pltpu.sync_copy(a12_hbm.at[h], a12_v)

    def _z1(i, _):
        amax_v[pl.ds(i * L, L)] = fzero
        return 0
    lax.fori_loop(0, NPAD // L, _z1, 0)

    def _z3(r, _):
        for c in range(HID2 // L):
            rows_v[r, pl.ds(c * L, L)] = fzero
        return 0
    lax.fori_loop(0, CH, _z3, 0)
    for b in range(SEG // CH):
        pltpu.sync_copy(rows_v,
                        _SH.out_sh.at[pl.ds(sid * SEG + b * CH, CH), :])

    def _p1(j, _):
        for i in range(VPC):
            sl = pl.ds(i * L, L)
            d_idx = dst_v[j, sl]
            s_idx = src_v[j, sl]
            x = (plsc.load_gather(a12_v, [d_idx * 2])
                 + plsc.load_gather(a12_v, [s_idx * 2 + one16]))
            a = jnp.where(x > 0, x, 0.01 * x)
            a_v[j, sl] = a
            old = plsc.load_gather(amax_v, [d_idx])
            plsc.store_scatter(amax_v, [d_idx], jnp.maximum(old, a))
            chk = plsc.load_gather(amax_v, [d_idx])
            pend = chk < a

            def _wcond(m):
                return jnp.any(m)

            def _wbody(m):
                o2 = plsc.load_gather(amax_v, [d_idx], mask=m)
                plsc.store_scatter(amax_v, [d_idx], jnp.maximum(o2, a),
                                   mask=m)
                c2 = plsc.load_gather(amax_v, [d_idx], mask=m)
                return (c2 < a) & m

            lax.while_loop(_wcond, _wbody, pend)
        return 0
    lax.fori_loop(0, NCHUNK, _p1, 0)

    seg_sl = pl.ds(sid * SEG, SEG)
    pltpu.sync_copy(amax_v, slab_hbm.at[cid, sid])
    plsc.subcore_barrier()
    pltpu.sync_copy(slab_hbm.at[cid, :, seg_sl], stage_v)

    def _red(i, _):
        sl = pl.ds(i * L, L)
        m = stage_v[0, sl]
        for r in range(1, NS):
            m = jnp.maximum(m, stage_v[r, sl])
        stage_v[0, sl] = m
        return 0
    lax.fori_loop(0, SEG // L, _red, 0)
    pltpu.sync_copy(stage_v.at[0], _SH.g_sh.at[seg_sl])
    plsc.subcore_barrier()
    pltpu.sync_copy(_SH.g_sh, amax_v)
    plsc.subcore_barrier()

    def _z2(i, _):
        stage_v[0, pl.ds(i * L, L)] = fzero
        return 0
    lax.fori_loop(0, SEG // L, _z2, 0)
    pltpu.sync_copy(stage_v.at[0], _SH.g_sh.at[seg_sl])
    plsc.subcore_barrier()

    def _p2(j, _):
        for i in range(VPC):
            sl = pl.ds(i * L, L)
            am = plsc.load_gather(amax_v, [dst_v[j, sl]])
            a_v[j, sl] = jnp.exp(a_v[j, sl] - am)
        pltpu.sync_copy(a_v.at[j], _SH.g_sh.at[src_v.at[j]], add=True)
        return 0
    lax.fori_loop(0, NCHUNK, _p2, 0)
    plsc.subcore_barrier()
    pltpu.sync_copy(_SH.g_sh, amax_v)

    def _p3(j, _):
        for i in range(VPC):
            sl = pl.ds(i * L, L)
            s = plsc.load_gather(amax_v, [dst_v[j, sl]])
            a_v[j, sl] = a_v[j, sl] / s
        pltpu.async_copy(ft_hbm.at[2 * h + cid].at[dst_v.at[j]],
                         rows_v, sem).wait()

        def _mul(g, _):
            av = a_v[j, pl.ds(g * L, L)]
            for r16 in range(L):
                r = g * L + r16
                t = jnp.broadcast_to(av[r16], (L,))
                for c in range(HID2 // L):
                    csl = pl.ds(c * L, L)
                    rows_v[r, csl] = rows_v[r, csl] * t
            return 0
        lax.fori_loop(0, CH // L, _mul, 0)
        pltpu.sync_copy(rows_v, _SH.out_sh.at[src_v.at[j]], add=True)
        return 0
    lax.fori_loop(0, NCHUNK, _p3, 0)

    plsc.subcore_barrier()
    pltpu.sync_copy(_SH.out_sh.at[pl.ds(sid * SEG, SEG), :],
                    outp.at[h, cid, sid])


def _sc_edge_body(nh, ft_hbm, a12_hbm, src_hbm, dst_hbm, outp, slab_hbm,
                  a12_v, amax_v, src_v, dst_v, a_v, rows_v, stage_v, sem):
    cid = lax.axis_index("c")
    sid = lax.axis_index("s")

    pltpu.sync_copy(src_hbm.at[sid], src_v)
    pltpu.sync_copy(dst_hbm.at[sid], dst_v)
    for h in range(nh):
        _sc_head(h, cid, sid, ft_hbm, a12_hbm, outp, slab_hbm,
                 a12_v, amax_v, src_v, dst_v, a_v, rows_v, stage_v, sem)


class _SHNS:
    g_sh = None
    out_sh = None


_SH = _SHNS()


def _make_wrapped(nh):
    def _sc_edge_wrapped(ft_hbm, a12_hbm, src_hbm, dst_hbm, outp, slab_hbm,
                         a12_v, amax_v, src_v, dst_v, a_v, rows_v, stage_v,
                         g_sh, out_sh, sem):
        _SH.g_sh, _SH.out_sh = g_sh, out_sh
        _sc_edge_body(nh, ft_hbm, a12_hbm, src_hbm, dst_hbm, outp, slab_hbm,
                      a12_v, amax_v, src_v, dst_v, a_v, rows_v, stage_v, sem)
    return _sc_edge_wrapped


def _sc_edge(ft, a12, src_rs, dst_rs):
    nh = a12.shape[0]
    mesh = plsc.VectorSubcoreMesh(core_axis_name="c", subcore_axis_name="s")
    f = functools.partial(
        pl.kernel,
        out_type=[
            jax.ShapeDtypeStruct((nh, NC, NS, SEG, HID2), jnp.float32),
            jax.ShapeDtypeStruct((NC, NS, NPAD), jnp.float32),
        ],
        mesh=mesh,
        compiler_params=pltpu.CompilerParams(
            needs_layout_passes=False, use_tc_tiling_on_sc=False),
        scratch_types=[
            pltpu.VMEM((2 * NPAD,), jnp.float32),
            pltpu.VMEM((NPAD,), jnp.float32),
            pltpu.VMEM((NCHUNK, CH), jnp.int32),
            pltpu.VMEM((NCHUNK, CH), jnp.int32),
            pltpu.VMEM((NCHUNK, CH), jnp.float32),
            pltpu.VMEM((CH, HID2), jnp.float32),
            pltpu.VMEM((NS, SEG), jnp.float32),
            pltpu.VMEM_SHARED((NPAD,), jnp.float32),
            pltpu.VMEM_SHARED((NPAD, HID2), jnp.float32),
            pltpu.SemaphoreType.DMA,
        ],
    )(_make_wrapped(nh))
    outp, _unused = f(ft, a12, src_rs, dst_rs)
    return outp


def kernel(features, src, dst, Wh, bh, alw, alb, arw, arb,
           Wo, bo, alow, alob, arow, arob):
    xp = jnp.zeros((NPAD, IN_DIM), jnp.float32).at[:N].set(features)
    src_rs = src.astype(jnp.int32).reshape(NS, NCHUNK, CH)
    dst_rs = dst.astype(jnp.int32).reshape(NS, NCHUNK, CH)

    bh3 = bh[:, None, :]
    aw2 = jnp.stack([alw, arw], axis=-1)
    ab2 = jnp.stack([alb, arb], axis=-1)[:, None, :]
    ft, a12 = _dense1(xp, Wh, bh3, aw2, ab2)

    outp1 = _sc_edge(ft.reshape(HEADS * NC, NPAD, HID2),
                     a12.reshape(HEADS, 2 * NPAD), src_rs, dst_rs)
    p0 = outp1[:, 0].reshape(HEADS, NPAD, HID2)
    p1 = outp1[:, 1].reshape(HEADS, NPAD, HID2)

    Wo3 = Wo.reshape(HEADS, HID, NCLS)
    Wo2a = Wo3[:, :HID2, :]
    Wo2b = Wo3[:, HID2:, :]
    bo2 = bo[None, :]
    aw22 = jnp.stack([alow, arow], axis=-1)
    ab22 = jnp.stack([alob, arob])[None, :]
    _ft2_full, ft2h, a12_2 = _dense2(p0, p1, Wo2a, Wo2b, bo2, aw22, ab22)

    outp2 = _sc_edge(ft2h, a12_2.reshape(1, 2 * NPAD), src_rs, dst_rs)
    q0 = outp2[0, 0].reshape(NPAD, HID2)
    q1 = outp2[0, 1].reshape(NPAD, HID2)
    return _final(q0, q1)[:N]


# speedup vs baseline: 33.9173x; 1.4041x over previous
import functools

import jax
import jax.numpy as jnp
from jax import lax
from jax.experimental import pallas as pl
from jax.experimental.pallas import tpu as pltpu
from jax.experimental.pallas import tpu_sc as plsc

N = 10000
E = 320000
IN_DIM = 128
HID = 64
HEADS = 4
NCLS = 64

NPAD = 10240
NC, NS, L = 2, 16, 16
HID2 = HID // 2
CH = 80
NCHUNK = (E // NS) // CH
SEG = NPAD // NS
VPC = CH // L
K2 = 2
K3 = 2


def _dense1_body(x_ref, w_ref, b_ref, aw_ref, ab_ref, ft_ref, a12_ref):
    ft = jnp.dot(x_ref[...], w_ref[0], preferred_element_type=jnp.float32)
    ft = ft + b_ref[0]
    ft_ref[0, 0] = ft[:, :HID2]
    ft_ref[0, 1] = ft[:, HID2:]
    a12 = jnp.dot(ft, aw_ref[0], preferred_element_type=jnp.float32)
    a12_ref[0] = a12 + ab_ref[0]


def _dense1(xp, Wh, bh3, aw2, ab2):
    return pl.pallas_call(
        _dense1_body,
        grid=(HEADS,),
        in_specs=[
            pl.BlockSpec((NPAD, IN_DIM), lambda h: (0, 0)),
            pl.BlockSpec((1, IN_DIM, HID), lambda h: (h, 0, 0)),
            pl.BlockSpec((1, 1, HID), lambda h: (h, 0, 0)),
            pl.BlockSpec((1, HID, 2), lambda h: (h, 0, 0)),
            pl.BlockSpec((1, 1, 2), lambda h: (h, 0, 0)),
        ],
        out_specs=[
            pl.BlockSpec((1, NC, NPAD, HID2), lambda h: (h, 0, 0, 0)),
            pl.BlockSpec((1, NPAD, 2), lambda h: (h, 0, 0)),
        ],
        out_shape=[
            jax.ShapeDtypeStruct((HEADS, NC, NPAD, HID2), jnp.float32),
            jax.ShapeDtypeStruct((HEADS, NPAD, 2), jnp.float32),
        ],
    )(xp, Wh, bh3, aw2, ab2)


def _dense2_body(p0_ref, p1_ref, wa_ref, wb_ref, b_ref, aw_ref, ab_ref,
                 ft2_ref, fth_ref, a12_ref):
    h = pl.program_id(0)
    s0 = p0_ref[0]
    s1 = p1_ref[0]
    h0 = jnp.where(s0 > 0, s0, jnp.exp(s0) - 1.0)
    h1 = jnp.where(s1 > 0, s1, jnp.exp(s1) - 1.0)
    part = (jnp.dot(h0, wa_ref[0], preferred_element_type=jnp.float32)
            + jnp.dot(h1, wb_ref[0], preferred_element_type=jnp.float32))

    @pl.when(h == 0)
    def _():
        ft2_ref[...] = part + b_ref[...]

    @pl.when(h > 0)
    def _():
        ft2_ref[...] = ft2_ref[...] + part

    @pl.when(h == HEADS - 1)
    def _():
        ft2 = ft2_ref[...]
        fth_ref[0] = ft2[:, :HID2]
        fth_ref[1] = ft2[:, HID2:]
        a12_ref[...] = (jnp.dot(ft2, aw_ref[...],
                                preferred_element_type=jnp.float32)
                        + ab_ref[...])


def _dense2(p0, p1, Wo2a, Wo2b, bo2, aw22, ab22):
    return pl.pallas_call(
        _dense2_body,
        grid=(HEADS,),
        in_specs=[
            pl.BlockSpec((1, NPAD, HID2), lambda h: (h, 0, 0)),
            pl.BlockSpec((1, NPAD, HID2), lambda h: (h, 0, 0)),
            pl.BlockSpec((1, HID2, NCLS), lambda h: (h, 0, 0)),
            pl.BlockSpec((1, HID2, NCLS), lambda h: (h, 0, 0)),
            pl.BlockSpec((1, NCLS), lambda h: (0, 0)),
            pl.BlockSpec((NCLS, 2), lambda h: (0, 0)),
            pl.BlockSpec((1, 2), lambda h: (0, 0)),
        ],
        out_specs=[
            pl.BlockSpec((NPAD, NCLS), lambda h: (0, 0)),
            pl.BlockSpec((NC, NPAD, HID2), lambda h: (0, 0, 0)),
            pl.BlockSpec((NPAD, 2), lambda h: (0, 0)),
        ],
        out_shape=[
            jax.ShapeDtypeStruct((NPAD, NCLS), jnp.float32),
            jax.ShapeDtypeStruct((NC, NPAD, HID2), jnp.float32),
            jax.ShapeDtypeStruct((NPAD, 2), jnp.float32),
        ],
    )(p0, p1, Wo2a, Wo2b, bo2, aw22, ab22)


def _final_body(q0_ref, q1_ref, o_ref):
    s0 = q0_ref[...]
    s1 = q1_ref[...]
    o_ref[:, :HID2] = jnp.where(s0 > 0, s0, jnp.exp(s0) - 1.0)
    o_ref[:, HID2:] = jnp.where(s1 > 0, s1, jnp.exp(s1) - 1.0)


def _final(q0, q1):
    return pl.pallas_call(
        _final_body,
        out_shape=jax.ShapeDtypeStruct((NPAD, NCLS), jnp.float32),
    )(q0, q1)


def _sc_head(h, cid, sid, ft_hbm, a12_hbm, outp, slab_hbm,
             a12_v, amax_v, src_v, dst_v, a_v, rows, stage_v,
             gsem, ssem, psem):
    one16 = jnp.ones((L,), jnp.int32)
    fzero = jnp.zeros((L,), jnp.float32)

    pltpu.sync_copy(a12_hbm.at[h], a12_v)

    def _z1(i, _):
        amax_v[pl.ds(i * L, L)] = fzero
        return 0
    lax.fori_loop(0, NPAD // L, _z1, 0)

    def _z3(r, _):
        for c in range(HID2 // L):
            rows[0][r, pl.ds(c * L, L)] = fzero
        return 0
    lax.fori_loop(0, CH, _z3, 0)
    for b in range(SEG // CH):
        pltpu.sync_copy(rows[0],
                        _SH.out_sh.at[pl.ds(sid * SEG + b * CH, CH), :])

    def _p1(j, _):
        for i in range(VPC):
            sl = pl.ds(i * L, L)
            d_idx = dst_v[j, sl]
            s_idx = src_v[j, sl]
            x = (plsc.load_gather(a12_v, [d_idx * 2])
                 + plsc.load_gather(a12_v, [s_idx * 2 + one16]))
            a = jnp.where(x > 0, x, 0.01 * x)
            a_v[j, sl] = a
            old = plsc.load_gather(amax_v, [d_idx])
            plsc.store_scatter(amax_v, [d_idx], jnp.maximum(old, a))
            chk = plsc.load_gather(amax_v, [d_idx])
            pend = chk < a

            def _wcond(m):
                return jnp.any(m)

            def _wbody(m):
                o2 = plsc.load_gather(amax_v, [d_idx], mask=m)
                plsc.store_scatter(amax_v, [d_idx], jnp.maximum(o2, a),
                                   mask=m)
                c2 = plsc.load_gather(amax_v, [d_idx], mask=m)
                return (c2 < a) & m

            lax.while_loop(_wcond, _wbody, pend)
        return 0
    lax.fori_loop(0, NCHUNK, _p1, 0)

    seg_sl = pl.ds(sid * SEG, SEG)
    pltpu.sync_copy(amax_v, slab_hbm.at[cid, sid])
    plsc.subcore_barrier()
    pltpu.sync_copy(slab_hbm.at[cid, :, seg_sl], stage_v)

    def _red(i, _):
        sl = pl.ds(i * L, L)
        m = stage_v[0, sl]
        for r in range(1, NS):
            m = jnp.maximum(m, stage_v[r, sl])
        stage_v[0, sl] = m
        return 0
    lax.fori_loop(0, SEG // L, _red, 0)
    pltpu.sync_copy(stage_v.at[0], _SH.g_sh.at[seg_sl])
    plsc.subcore_barrier()
    pltpu.sync_copy(_SH.g_sh, amax_v)
    plsc.subcore_barrier()

    def _z2(i, _):
        stage_v[0, pl.ds(i * L, L)] = fzero
        return 0
    lax.fori_loop(0, SEG // L, _z2, 0)
    pltpu.sync_copy(stage_v.at[0], _SH.g_sh.at[seg_sl])
    plsc.subcore_barrier()

    def _p2g(g, _):
        descs = []
        for q in range(K2):
            j = g * K2 + q
            for i in range(VPC):
                sl = pl.ds(i * L, L)
                am = plsc.load_gather(amax_v, [dst_v[j, sl]])
                a_v[j, sl] = jnp.exp(a_v[j, sl] - am)
            descs.append(pltpu.async_copy(
                a_v.at[j], _SH.g_sh.at[src_v.at[j]], psem, add=True))
        for d in descs:
            d.wait()
        return 0
    lax.fori_loop(0, NCHUNK // K2, _p2g, 0)
    plsc.subcore_barrier()
    pltpu.sync_copy(_SH.g_sh, amax_v)

    def _p3g(g, _):
        gd = []
        for q in range(K3):
            j = g * K3 + q
            gd.append(pltpu.async_copy(
                ft_hbm.at[2 * h + cid].at[dst_v.at[j]], rows[q], gsem))
        sd = []
        for q in range(K3):
            j = g * K3 + q
            for i in range(VPC):
                sl = pl.ds(i * L, L)
                s = plsc.load_gather(amax_v, [dst_v[j, sl]])
                a_v[j, sl] = a_v[j, sl] / s
            gd[q].wait()
            rq = rows[q]

            def _mul(gg, _, j=j, rq=rq):
                av = a_v[j, pl.ds(gg * L, L)]
                for r16 in range(L):
                    r = gg * L + r16
                    t = jnp.broadcast_to(av[r16], (L,))
                    for c in range(HID2 // L):
                        csl = pl.ds(c * L, L)
                        rq[r, csl] = rq[r, csl] * t
                return 0
            lax.fori_loop(0, CH // L, _mul, 0)
            sd.append(pltpu.async_copy(
                rq, _SH.out_sh.at[src_v.at[j]], ssem, add=True))
        for d in sd:
            d.wait()
        return 0
    lax.fori_loop(0, NCHUNK // K3, _p3g, 0)

    plsc.subcore_barrier()
    pltpu.sync_copy(_SH.out_sh.at[pl.ds(sid * SEG, SEG), :],
                    outp.at[h, cid, sid])


def _sc_edge_body(nh, ft_hbm, a12_hbm, src_hbm, dst_hbm, outp, slab_hbm,
                  a12_v, amax_v, src_v, dst_v, a_v, rows, stage_v,
                  gsem, ssem, psem):
    cid = lax.axis_index("c")
    sid = lax.axis_index("s")

    pltpu.sync_copy(src_hbm.at[sid], src_v)
    pltpu.sync_copy(dst_hbm.at[sid], dst_v)
    for h in range(nh):
        _sc_head(h, cid, sid, ft_hbm, a12_hbm, outp, slab_hbm,
                 a12_v, amax_v, src_v, dst_v, a_v, rows, stage_v,
                 gsem, ssem, psem)


class _SHNS:
    g_sh = None
    out_sh = None


_SH = _SHNS()


def _make_wrapped(nh):
    def _sc_edge_wrapped(ft_hbm, a12_hbm, src_hbm, dst_hbm, outp, slab_hbm,
                         a12_v, amax_v, src_v, dst_v, a_v, rows, stage_v,
                         g_sh, out_sh, gsem, ssem, psem):
        _SH.g_sh, _SH.out_sh = g_sh, out_sh
        _sc_edge_body(nh, ft_hbm, a12_hbm, src_hbm, dst_hbm, outp, slab_hbm,
                      a12_v, amax_v, src_v, dst_v, a_v, rows, stage_v,
                      gsem, ssem, psem)
    return _sc_edge_wrapped


def _sc_edge(ft, a12, src_rs, dst_rs):
    nh = a12.shape[0]
    mesh = plsc.VectorSubcoreMesh(core_axis_name="c", subcore_axis_name="s")
    f = functools.partial(
        pl.kernel,
        out_type=[
            jax.ShapeDtypeStruct((nh, NC, NS, SEG, HID2), jnp.float32),
            jax.ShapeDtypeStruct((NC, NS, NPAD), jnp.float32),
        ],
        mesh=mesh,
        compiler_params=pltpu.CompilerParams(
            needs_layout_passes=False, use_tc_tiling_on_sc=False),
        scratch_types=[
            pltpu.VMEM((2 * NPAD,), jnp.float32),
            pltpu.VMEM((NPAD,), jnp.float32),
            pltpu.VMEM((NCHUNK, CH), jnp.int32),
            pltpu.VMEM((NCHUNK, CH), jnp.int32),
            pltpu.VMEM((NCHUNK, CH), jnp.float32),
            [pltpu.VMEM((CH, HID2), jnp.float32)] * K3,
            pltpu.VMEM((NS, SEG), jnp.float32),
            pltpu.VMEM_SHARED((NPAD,), jnp.float32),
            pltpu.VMEM_SHARED((NPAD, HID2), jnp.float32),
            pltpu.SemaphoreType.DMA,
            pltpu.SemaphoreType.DMA,
            pltpu.SemaphoreType.DMA,
        ],
    )(_make_wrapped(nh))
    outp, _unused = f(ft, a12, src_rs, dst_rs)
    return outp


def kernel(features, src, dst, Wh, bh, alw, alb, arw, arb,
           Wo, bo, alow, alob, arow, arob):
    xp = jnp.zeros((NPAD, IN_DIM), jnp.float32).at[:N].set(features)
    src_rs = src.astype(jnp.int32).reshape(NS, NCHUNK, CH)
    dst_rs = dst.astype(jnp.int32).reshape(NS, NCHUNK, CH)

    bh3 = bh[:, None, :]
    aw2 = jnp.stack([alw, arw], axis=-1)
    ab2 = jnp.stack([alb, arb], axis=-1)[:, None, :]
    ft, a12 = _dense1(xp, Wh, bh3, aw2, ab2)

    outp1 = _sc_edge(ft.reshape(HEADS * NC, NPAD, HID2),
                     a12.reshape(HEADS, 2 * NPAD), src_rs, dst_rs)
    p0 = outp1[:, 0].reshape(HEADS, NPAD, HID2)
    p1 = outp1[:, 1].reshape(HEADS, NPAD, HID2)

    Wo3 = Wo.reshape(HEADS, HID, NCLS)
    Wo2a = Wo3[:, :HID2, :]
    Wo2b = Wo3[:, HID2:, :]
    bo2 = bo[None, :]
    aw22 = jnp.stack([alow, arow], axis=-1)
    ab22 = jnp.stack([alob, arob])[None, :]
    _ft2_full, ft2h, a12_2 = _dense2(p0, p1, Wo2a, Wo2b, bo2, aw22, ab22)

    outp2 = _sc_edge(ft2h, a12_2.reshape(1, 2 * NPAD), src_rs, dst_rs)
    q0 = outp2[0, 0].reshape(NPAD, HID2)
    q1 = outp2[0, 1].reshape(NPAD, HID2)
    return _final(q0, q1)[:N]


# speedup vs baseline: 42.7626x; 1.2608x over previous
import functools

import jax
import jax.numpy as jnp
from jax import lax
from jax.experimental import pallas as pl
from jax.experimental.pallas import tpu as pltpu
from jax.experimental.pallas import tpu_sc as plsc

N = 10000
E = 320000
IN_DIM = 128
HID = 64
HEADS = 4
NCLS = 64

NPAD = 10240
NC, NS, L = 2, 16, 16
HID2 = HID // 2
CH = 80
NCHUNK = (E // NS) // CH
SEG = NPAD // NS
VPC = CH // L
K2 = 5
K3 = 5


def _dense1_body(x_ref, w_ref, b_ref, aw_ref, ab_ref, ft_ref, a12_ref):
    ft = jnp.dot(x_ref[...], w_ref[0], preferred_element_type=jnp.float32)
    ft = ft + b_ref[0]
    ft_ref[0, 0] = ft[:, :HID2]
    ft_ref[0, 1] = ft[:, HID2:]
    a12 = jnp.dot(ft, aw_ref[0], preferred_element_type=jnp.float32)
    a12_ref[0] = a12 + ab_ref[0]


def _dense1(xp, Wh, bh3, aw2, ab2):
    return pl.pallas_call(
        _dense1_body,
        grid=(HEADS,),
        in_specs=[
            pl.BlockSpec((NPAD, IN_DIM), lambda h: (0, 0)),
            pl.BlockSpec((1, IN_DIM, HID), lambda h: (h, 0, 0)),
            pl.BlockSpec((1, 1, HID), lambda h: (h, 0, 0)),
            pl.BlockSpec((1, HID, 2), lambda h: (h, 0, 0)),
            pl.BlockSpec((1, 1, 2), lambda h: (h, 0, 0)),
        ],
        out_specs=[
            pl.BlockSpec((1, NC, NPAD, HID2), lambda h: (h, 0, 0, 0)),
            pl.BlockSpec((1, NPAD, 2), lambda h: (h, 0, 0)),
        ],
        out_shape=[
            jax.ShapeDtypeStruct((HEADS, NC, NPAD, HID2), jnp.float32),
            jax.ShapeDtypeStruct((HEADS, NPAD, 2), jnp.float32),
        ],
    )(xp, Wh, bh3, aw2, ab2)


def _dense2_body(p0_ref, p1_ref, wa_ref, wb_ref, b_ref, aw_ref, ab_ref,
                 ft2_ref, fth_ref, a12_ref):
    h = pl.program_id(0)
    s0 = p0_ref[0]
    s1 = p1_ref[0]
    h0 = jnp.where(s0 > 0, s0, jnp.exp(s0) - 1.0)
    h1 = jnp.where(s1 > 0, s1, jnp.exp(s1) - 1.0)
    part = (jnp.dot(h0, wa_ref[0], preferred_element_type=jnp.float32)
            + jnp.dot(h1, wb_ref[0], preferred_element_type=jnp.float32))

    @pl.when(h == 0)
    def _():
        ft2_ref[...] = part + b_ref[...]

    @pl.when(h > 0)
    def _():
        ft2_ref[...] = ft2_ref[...] + part

    @pl.when(h == HEADS - 1)
    def _():
        ft2 = ft2_ref[...]
        fth_ref[0] = ft2[:, :HID2]
        fth_ref[1] = ft2[:, HID2:]
        a12_ref[...] = (jnp.dot(ft2, aw_ref[...],
                                preferred_element_type=jnp.float32)
                        + ab_ref[...])


def _dense2(p0, p1, Wo2a, Wo2b, bo2, aw22, ab22):
    return pl.pallas_call(
        _dense2_body,
        grid=(HEADS,),
        in_specs=[
            pl.BlockSpec((1, NPAD, HID2), lambda h: (h, 0, 0)),
            pl.BlockSpec((1, NPAD, HID2), lambda h: (h, 0, 0)),
            pl.BlockSpec((1, HID2, NCLS), lambda h: (h, 0, 0)),
            pl.BlockSpec((1, HID2, NCLS), lambda h: (h, 0, 0)),
            pl.BlockSpec((1, NCLS), lambda h: (0, 0)),
            pl.BlockSpec((NCLS, 2), lambda h: (0, 0)),
            pl.BlockSpec((1, 2), lambda h: (0, 0)),
        ],
        out_specs=[
            pl.BlockSpec((NPAD, NCLS), lambda h: (0, 0)),
            pl.BlockSpec((NC, NPAD, HID2), lambda h: (0, 0, 0)),
            pl.BlockSpec((NPAD, 2), lambda h: (0, 0)),
        ],
        out_shape=[
            jax.ShapeDtypeStruct((NPAD, NCLS), jnp.float32),
            jax.ShapeDtypeStruct((NC, NPAD, HID2), jnp.float32),
            jax.ShapeDtypeStruct((NPAD, 2), jnp.float32),
        ],
    )(p0, p1, Wo2a, Wo2b, bo2, aw22, ab22)


def _sc_head(h, cid, sid, ft_hbm, a12_hbm, outp, slab_hbm,
             a12_v, amax_v, src_v, dst_v, a_v, rows, stage_v,
             gsem, ssem, psem, do_elu):
    one16 = jnp.ones((L,), jnp.int32)
    fzero = jnp.zeros((L,), jnp.float32)

    pltpu.sync_copy(a12_hbm.at[h], a12_v)

    def _z1(i, _):
        amax_v[pl.ds(i * L, L)] = fzero
        return 0
    lax.fori_loop(0, NPAD // L, _z1, 0)

    def _z3(r, _):
        for c in range(HID2 // L):
            rows[0][r, pl.ds(c * L, L)] = fzero
        return 0
    lax.fori_loop(0, CH, _z3, 0)
    zd = [pltpu.async_copy(rows[0],
                           _SH.out_sh.at[pl.ds(sid * SEG + b * CH, CH), :],
                           gsem)
          for b in range(SEG // CH)]

    def _p1(j, _):
        for i in range(VPC):
            sl = pl.ds(i * L, L)
            d_idx = dst_v[j, sl]
            s_idx = src_v[j, sl]
            x = (plsc.load_gather(a12_v, [d_idx * 2])
                 + plsc.load_gather(a12_v, [s_idx * 2 + one16]))
            a = jnp.where(x > 0, x, 0.01 * x)
            a_v[j, sl] = a
            old = plsc.load_gather(amax_v, [d_idx])
            plsc.store_scatter(amax_v, [d_idx], jnp.maximum(old, a))
            chk = plsc.load_gather(amax_v, [d_idx])
            pend = chk < a

            def _wcond(m):
                return jnp.any(m)

            def _wbody(m):
                o2 = plsc.load_gather(amax_v, [d_idx], mask=m)
                plsc.store_scatter(amax_v, [d_idx], jnp.maximum(o2, a),
                                   mask=m)
                c2 = plsc.load_gather(amax_v, [d_idx], mask=m)
                return (c2 < a) & m

            lax.while_loop(_wcond, _wbody, pend)
        return 0
    lax.fori_loop(0, NCHUNK, _p1, 0)

    HSEG = SEG // 2
    pltpu.sync_copy(amax_v, slab_hbm.at[cid, sid])
    plsc.subcore_barrier()
    for half in range(2):
        hs = pl.ds(sid * SEG + half * HSEG, HSEG)
        pltpu.sync_copy(slab_hbm.at[cid, :, hs], stage_v)

        def _red(i, _):
            sl = pl.ds(i * L, L)
            m = stage_v[0, sl]
            for r in range(1, NS):
                m = jnp.maximum(m, stage_v[r, sl])
            stage_v[0, sl] = m
            return 0
        lax.fori_loop(0, HSEG // L, _red, 0)
        pltpu.sync_copy(stage_v.at[0], _SH.g_sh.at[hs])
    plsc.subcore_barrier()
    pltpu.sync_copy(_SH.g_sh, amax_v)
    plsc.subcore_barrier()

    def _z2(i, _):
        stage_v[0, pl.ds(i * L, L)] = fzero
        return 0
    lax.fori_loop(0, HSEG // L, _z2, 0)
    for half in range(2):
        pltpu.sync_copy(stage_v.at[0],
                        _SH.g_sh.at[pl.ds(sid * SEG + half * HSEG, HSEG)])
    plsc.subcore_barrier()

    def _p2g(g, _):
        descs = []
        for q in range(K2):
            j = g * K2 + q
            for i in range(VPC):
                sl = pl.ds(i * L, L)
                am = plsc.load_gather(amax_v, [dst_v[j, sl]])
                a_v[j, sl] = jnp.exp(a_v[j, sl] - am)
            descs.append(pltpu.async_copy(
                a_v.at[j], _SH.g_sh.at[src_v.at[j]], psem, add=True))
        for d in descs:
            d.wait()
        return 0
    lax.fori_loop(0, NCHUNK // K2, _p2g, 0)
    for d in zd:
        d.wait()
    plsc.subcore_barrier()
    pltpu.sync_copy(_SH.g_sh, amax_v)

    def _inv(i, _):
        sl = pl.ds(i * L, L)
        amax_v[sl] = 1.0 / amax_v[sl]
        return 0
    lax.fori_loop(0, NPAD // L, _inv, 0)

    def _p3g(g, _):
        gd = []
        for q in range(K3):
            j = g * K3 + q
            gd.append(pltpu.async_copy(
                ft_hbm.at[2 * h + cid].at[dst_v.at[j]], rows[q], gsem))
        sd = []
        for q in range(K3):
            j = g * K3 + q
            for i in range(VPC):
                sl = pl.ds(i * L, L)
                s = plsc.load_gather(amax_v, [dst_v[j, sl]])
                a_v[j, sl] = a_v[j, sl] * s
            gd[q].wait()
            rq = rows[q]

            def _mul(gg, _, j=j, rq=rq):
                av = a_v[j, pl.ds(gg * L, L)]
                for r16 in range(L):
                    r = gg * L + r16
                    t = jnp.broadcast_to(av[r16], (L,))
                    for c in range(HID2 // L):
                        csl = pl.ds(c * L, L)
                        rq[r, csl] = rq[r, csl] * t
                return 0
            lax.fori_loop(0, CH // L, _mul, 0)
            sd.append(pltpu.async_copy(
                rq, _SH.out_sh.at[src_v.at[j]], ssem, add=True))
        for d in sd:
            d.wait()
        return 0
    lax.fori_loop(0, NCHUNK // K3, _p3g, 0)

    plsc.subcore_barrier()
    if not do_elu:
        pltpu.sync_copy(_SH.out_sh.at[pl.ds(sid * SEG, SEG), :],
                        outp.at[h, cid, sid])
    else:
        nb = SEG // CH
        ind = {}
        outd = {}
        for b in range(min(K3, nb)):
            ind[b] = pltpu.async_copy(
                _SH.out_sh.at[pl.ds(sid * SEG + b * CH, CH), :],
                rows[b], gsem)
        for b in range(nb):
            q = b % K3
            ind[b].wait()

            def _elu(r, _, q=q):
                for c in range(HID2 // L):
                    csl = pl.ds(c * L, L)
                    v = rows[q][r, csl]
                    rows[q][r, csl] = jnp.where(v > 0, v, jnp.exp(v) - 1.0)
                return 0
            lax.fori_loop(0, CH, _elu, 0)
            outd[b] = pltpu.async_copy(
                rows[q], outp.at[h, cid, sid].at[pl.ds(b * CH, CH), :],
                ssem)
            nxt = b + K3
            if nxt < nb:
                outd[b].wait()
                del outd[b]
                ind[nxt] = pltpu.async_copy(
                    _SH.out_sh.at[pl.ds(sid * SEG + nxt * CH, CH), :],
                    rows[q], gsem)
        for b in sorted(outd):
            outd[b].wait()


def _sc_edge_body(nh, do_elu, ft_hbm, a12_hbm, src_hbm, dst_hbm, outp,
                  slab_hbm, a12_v, amax_v, src_v, dst_v, a_v, rows, stage_v,
                  gsem, ssem, psem):
    cid = lax.axis_index("c")
    sid = lax.axis_index("s")

    pltpu.sync_copy(src_hbm.at[sid], src_v)
    pltpu.sync_copy(dst_hbm.at[sid], dst_v)
    for h in range(nh):
        _sc_head(h, cid, sid, ft_hbm, a12_hbm, outp, slab_hbm,
                 a12_v, amax_v, src_v, dst_v, a_v, rows, stage_v,
                 gsem, ssem, psem, do_elu)


class _SHNS:
    g_sh = None
    out_sh = None


_SH = _SHNS()


def _make_wrapped(nh, do_elu):
    def _sc_edge_wrapped(ft_hbm, a12_hbm, src_hbm, dst_hbm, outp, slab_hbm,
                         a12_v, amax_v, src_v, dst_v, a_v, rows, stage_v,
                         g_sh, out_sh, gsem, ssem, psem):
        _SH.g_sh, _SH.out_sh = g_sh, out_sh
        _sc_edge_body(nh, do_elu, ft_hbm, a12_hbm, src_hbm, dst_hbm, outp,
                      slab_hbm, a12_v, amax_v, src_v, dst_v, a_v, rows,
                      stage_v, gsem, ssem, psem)
    return _sc_edge_wrapped


def _sc_edge(ft, a12, src_rs, dst_rs, do_elu=False):
    nh = a12.shape[0]
    mesh = plsc.VectorSubcoreMesh(core_axis_name="c", subcore_axis_name="s")
    f = functools.partial(
        pl.kernel,
        out_type=[
            jax.ShapeDtypeStruct((nh, NC, NS, SEG, HID2), jnp.float32),
            jax.ShapeDtypeStruct((NC, NS, NPAD), jnp.float32),
        ],
        mesh=mesh,
        compiler_params=pltpu.CompilerParams(
            needs_layout_passes=False, use_tc_tiling_on_sc=False),
        scratch_types=[
            pltpu.VMEM((2 * NPAD,), jnp.float32),
            pltpu.VMEM((NPAD,), jnp.float32),
            pltpu.VMEM((NCHUNK, CH), jnp.int32),
            pltpu.VMEM((NCHUNK, CH), jnp.int32),
            pltpu.VMEM((NCHUNK, CH), jnp.float32),
            [pltpu.VMEM((CH, HID2), jnp.float32)] * K3,
            pltpu.VMEM((NS, SEG // 2), jnp.float32),
            pltpu.VMEM_SHARED((NPAD,), jnp.float32),
            pltpu.VMEM_SHARED((NPAD, HID2), jnp.float32),
            pltpu.SemaphoreType.DMA,
            pltpu.SemaphoreType.DMA,
            pltpu.SemaphoreType.DMA,
        ],
    )(_make_wrapped(nh, do_elu))
    outp, _unused = f(ft, a12, src_rs, dst_rs)
    return outp


def kernel(features, src, dst, Wh, bh, alw, alb, arw, arb,
           Wo, bo, alow, alob, arow, arob):
    xp = jnp.zeros((NPAD, IN_DIM), jnp.float32).at[:N].set(features)
    src_rs = src.astype(jnp.int32).reshape(NS, NCHUNK, CH)
    dst_rs = dst.astype(jnp.int32).reshape(NS, NCHUNK, CH)

    bh3 = bh[:, None, :]
    aw2 = jnp.stack([alw, arw], axis=-1)
    ab2 = jnp.stack([alb, arb], axis=-1)[:, None, :]
    ft, a12 = _dense1(xp, Wh, bh3, aw2, ab2)

    outp1 = _sc_edge(ft.reshape(HEADS * NC, NPAD, HID2),
                     a12.reshape(HEADS, 2 * NPAD), src_rs, dst_rs)
    p0 = outp1[:, 0].reshape(HEADS, NPAD, HID2)
    p1 = outp1[:, 1].reshape(HEADS, NPAD, HID2)

    Wo3 = Wo.reshape(HEADS, HID, NCLS)
    Wo2a = Wo3[:, :HID2, :]
    Wo2b = Wo3[:, HID2:, :]
    bo2 = bo[None, :]
    aw22 = jnp.stack([alow, arow], axis=-1)
    ab22 = jnp.stack([alob, arob])[None, :]
    _ft2_full, ft2h, a12_2 = _dense2(p0, p1, Wo2a, Wo2b, bo2, aw22, ab22)

    outp2 = _sc_edge(ft2h, a12_2.reshape(1, 2 * NPAD), src_rs, dst_rs,
                     do_elu=True)
    q0 = outp2[0, 0].reshape(NPAD, HID2)
    q1 = outp2[0, 1].reshape(NPAD, HID2)
    return jnp.concatenate([q0, q1], axis=1)[:N]


# speedup vs baseline: 42.8977x; 1.0032x over previous
import functools

import jax
import jax.numpy as jnp
from jax import lax
from jax.experimental import pallas as pl
from jax.experimental.pallas import tpu as pltpu
from jax.experimental.pallas import tpu_sc as plsc

N = 10000
E = 320000
IN_DIM = 128
HID = 64
HEADS = 4
NCLS = 64

NPAD = 10240
NC, NS, L = 2, 16, 16
HID2 = HID // 2
CH = 80
NCHUNK = (E // NS) // CH
SEG = NPAD // NS
VPC = CH // L
K2 = 5
K3 = 5


def _dense1_body(x_ref, w_ref, b_ref, aw_ref, ab_ref, ft_ref, a12_ref):
    ft = jnp.dot(x_ref[...], w_ref[0], preferred_element_type=jnp.float32)
    ft = ft + b_ref[0]
    ft_ref[0, 0, pl.ds(0, N)] = ft[:, :HID2]
    ft_ref[0, 1, pl.ds(0, N)] = ft[:, HID2:]
    a12 = jnp.dot(ft, aw_ref[0], preferred_element_type=jnp.float32)
    a12_ref[0, pl.ds(0, N)] = a12 + ab_ref[0]


def _dense1(x, Wh, bh3, aw2, ab2):
    return pl.pallas_call(
        _dense1_body,
        grid=(HEADS,),
        in_specs=[
            pl.BlockSpec((N, IN_DIM), lambda h: (0, 0)),
            pl.BlockSpec((1, IN_DIM, HID), lambda h: (h, 0, 0)),
            pl.BlockSpec((1, 1, HID), lambda h: (h, 0, 0)),
            pl.BlockSpec((1, HID, 2), lambda h: (h, 0, 0)),
            pl.BlockSpec((1, 1, 2), lambda h: (h, 0, 0)),
        ],
        out_specs=[
            pl.BlockSpec((1, NC, NPAD, HID2), lambda h: (h, 0, 0, 0)),
            pl.BlockSpec((1, NPAD, 2), lambda h: (h, 0, 0)),
        ],
        out_shape=[
            jax.ShapeDtypeStruct((HEADS, NC, NPAD, HID2), jnp.float32),
            jax.ShapeDtypeStruct((HEADS, NPAD, 2), jnp.float32),
        ],
    )(x, Wh, bh3, aw2, ab2)


def _dense2_body(p0_ref, p1_ref, wa_ref, wb_ref, b_ref, aw_ref, ab_ref,
                 ft2_ref, fth_ref, a12_ref):
    h = pl.program_id(0)
    s0 = p0_ref[0]
    s1 = p1_ref[0]
    h0 = jnp.where(s0 > 0, s0, jnp.exp(s0) - 1.0)
    h1 = jnp.where(s1 > 0, s1, jnp.exp(s1) - 1.0)
    part = (jnp.dot(h0, wa_ref[0], preferred_element_type=jnp.float32)
            + jnp.dot(h1, wb_ref[0], preferred_element_type=jnp.float32))

    @pl.when(h == 0)
    def _():
        ft2_ref[...] = part + b_ref[...]

    @pl.when(h > 0)
    def _():
        ft2_ref[...] = ft2_ref[...] + part

    @pl.when(h == HEADS - 1)
    def _():
        ft2 = ft2_ref[...]
        fth_ref[0] = ft2[:, :HID2]
        fth_ref[1] = ft2[:, HID2:]
        a12_ref[...] = (jnp.dot(ft2, aw_ref[...],
                                preferred_element_type=jnp.float32)
                        + ab_ref[...])


def _dense2(p0, p1, Wo2a, Wo2b, bo2, aw22, ab22):
    return pl.pallas_call(
        _dense2_body,
        grid=(HEADS,),
        in_specs=[
            pl.BlockSpec((1, NPAD, HID2), lambda h: (h, 0, 0)),
            pl.BlockSpec((1, NPAD, HID2), lambda h: (h, 0, 0)),
            pl.BlockSpec((1, HID2, NCLS), lambda h: (h, 0, 0)),
            pl.BlockSpec((1, HID2, NCLS), lambda h: (h, 0, 0)),
            pl.BlockSpec((1, NCLS), lambda h: (0, 0)),
            pl.BlockSpec((NCLS, 2), lambda h: (0, 0)),
            pl.BlockSpec((1, 2), lambda h: (0, 0)),
        ],
        out_specs=[
            pl.BlockSpec((NPAD, NCLS), lambda h: (0, 0)),
            pl.BlockSpec((NC, NPAD, HID2), lambda h: (0, 0, 0)),
            pl.BlockSpec((NPAD, 2), lambda h: (0, 0)),
        ],
        out_shape=[
            jax.ShapeDtypeStruct((NPAD, NCLS), jnp.float32),
            jax.ShapeDtypeStruct((NC, NPAD, HID2), jnp.float32),
            jax.ShapeDtypeStruct((NPAD, 2), jnp.float32),
        ],
    )(p0, p1, Wo2a, Wo2b, bo2, aw22, ab22)


def _sc_head(h, cid, sid, ft_hbm, a12_hbm, outp, slab_hbm,
             a12_v, amax_v, src_v, dst_v, a_v, rows, stage_v,
             gsem, ssem, psem, do_elu):
    one16 = jnp.ones((L,), jnp.int32)
    fzero = jnp.zeros((L,), jnp.float32)

    pltpu.sync_copy(a12_hbm.at[h], a12_v)

    def _z1(i, _):
        amax_v[pl.ds(i * L, L)] = fzero
        return 0
    lax.fori_loop(0, NPAD // L, _z1, 0)

    def _z3(r, _):
        for c in range(HID2 // L):
            rows[0][r, pl.ds(c * L, L)] = fzero
        return 0
    lax.fori_loop(0, CH, _z3, 0)
    zd = [pltpu.async_copy(rows[0],
                           _SH.out_sh.at[pl.ds(sid * SEG + b * CH, CH), :],
                           gsem)
          for b in range(SEG // CH)]

    def _p1(j, _):
        for i in range(VPC):
            sl = pl.ds(i * L, L)
            d_idx = dst_v[j, sl]
            s_idx = src_v[j, sl]
            x = (plsc.load_gather(a12_v, [d_idx * 2])
                 + plsc.load_gather(a12_v, [s_idx * 2 + one16]))
            a = jnp.where(x > 0, x, 0.01 * x)
            a_v[j, sl] = a
            old = plsc.load_gather(amax_v, [d_idx])
            plsc.store_scatter(amax_v, [d_idx], jnp.maximum(old, a))
            chk = plsc.load_gather(amax_v, [d_idx])
            pend = chk < a

            def _wcond(m):
                return jnp.any(m)

            def _wbody(m):
                o2 = plsc.load_gather(amax_v, [d_idx], mask=m)
                plsc.store_scatter(amax_v, [d_idx], jnp.maximum(o2, a),
                                   mask=m)
                c2 = plsc.load_gather(amax_v, [d_idx], mask=m)
                return (c2 < a) & m

            lax.while_loop(_wcond, _wbody, pend)
        return 0
    lax.fori_loop(0, NCHUNK, _p1, 0)

    HSEG = SEG // 2
    pltpu.sync_copy(amax_v, slab_hbm.at[cid, sid])
    plsc.subcore_barrier()
    for half in range(2):
        hs = pl.ds(sid * SEG + half * HSEG, HSEG)
        pltpu.sync_copy(slab_hbm.at[cid, :, hs], stage_v)

        def _red(i, _):
            sl = pl.ds(i * L, L)
            m = stage_v[0, sl]
            for r in range(1, NS):
                m = jnp.maximum(m, stage_v[r, sl])
            stage_v[0, sl] = m
            return 0
        lax.fori_loop(0, HSEG // L, _red, 0)
        pltpu.sync_copy(stage_v.at[0], _SH.g_sh.at[hs])
    plsc.subcore_barrier()
    pltpu.sync_copy(_SH.g_sh, amax_v)
    plsc.subcore_barrier()

    def _z2(i, _):
        stage_v[0, pl.ds(i * L, L)] = fzero
        return 0
    lax.fori_loop(0, HSEG // L, _z2, 0)
    for half in range(2):
        pltpu.sync_copy(stage_v.at[0],
                        _SH.g_sh.at[pl.ds(sid * SEG + half * HSEG, HSEG)])
    plsc.subcore_barrier()

    def _p2g(g, _):
        descs = []
        for q in range(K2):
            j = g * K2 + q
            for i in range(VPC):
                sl = pl.ds(i * L, L)
                am = plsc.load_gather(amax_v, [dst_v[j, sl]])
                a_v[j, sl] = jnp.exp(a_v[j, sl] - am)
            descs.append(pltpu.async_copy(
                a_v.at[j], _SH.g_sh.at[src_v.at[j]], psem, add=True))
        for d in descs:
            d.wait()
        return 0
    lax.fori_loop(0, NCHUNK // K2, _p2g, 0)
    for d in zd:
        d.wait()
    plsc.subcore_barrier()
    pltpu.sync_copy(_SH.g_sh, amax_v)

    def _inv(i, _):
        sl = pl.ds(i * L, L)
        amax_v[sl] = 1.0 / amax_v[sl]
        return 0
    lax.fori_loop(0, NPAD // L, _inv, 0)

    def _p3g(g, _):
        gd = []
        for q in range(K3):
            j = g * K3 + q
            gd.append(pltpu.async_copy(
                ft_hbm.at[2 * h + cid].at[dst_v.at[j]], rows[q], gsem))
        sd = []
        for q in range(K3):
            j = g * K3 + q
            for i in range(VPC):
                sl = pl.ds(i * L, L)
                s = plsc.load_gather(amax_v, [dst_v[j, sl]])
                a_v[j, sl] = a_v[j, sl] * s
            gd[q].wait()
            rq = rows[q]

            def _mul(gg, _, j=j, rq=rq):
                av = a_v[j, pl.ds(gg * L, L)]
                for r16 in range(L):
                    r = gg * L + r16
                    t = jnp.broadcast_to(av[r16], (L,))
                    for c in range(HID2 // L):
                        csl = pl.ds(c * L, L)
                        rq[r, csl] = rq[r, csl] * t
                return 0
            lax.fori_loop(0, CH // L, _mul, 0)
            sd.append(pltpu.async_copy(
                rq, _SH.out_sh.at[src_v.at[j]], ssem, add=True))
        for d in sd:
            d.wait()
        return 0
    lax.fori_loop(0, NCHUNK // K3, _p3g, 0)

    plsc.subcore_barrier()
    if not do_elu:
        pltpu.sync_copy(_SH.out_sh.at[pl.ds(sid * SEG, SEG), :],
                        outp.at[cid, h, sid])
    else:
        nb = SEG // CH
        ind = {}
        outd = {}
        for b in range(min(K3, nb)):
            ind[b] = pltpu.async_copy(
                _SH.out_sh.at[pl.ds(sid * SEG + b * CH, CH), :],
                rows[b], gsem)
        for b in range(nb):
            q = b % K3
            ind[b].wait()

            def _elu(r, _, q=q):
                for c in range(HID2 // L):
                    csl = pl.ds(c * L, L)
                    v = rows[q][r, csl]
                    rows[q][r, csl] = jnp.where(v > 0, v, jnp.exp(v) - 1.0)
                return 0
            lax.fori_loop(0, CH, _elu, 0)
            outd[b] = pltpu.async_copy(
                rows[q], outp.at[cid, h, sid].at[pl.ds(b * CH, CH), :],
                ssem)
            nxt = b + K3
            if nxt < nb:
                outd[b].wait()
                del outd[b]
                ind[nxt] = pltpu.async_copy(
                    _SH.out_sh.at[pl.ds(sid * SEG + nxt * CH, CH), :],
                    rows[q], gsem)
        for b in sorted(outd):
            outd[b].wait()


def _sc_edge_body(nh, do_elu, ft_hbm, a12_hbm, src_hbm, dst_hbm, outp,
                  slab_hbm, a12_v, amax_v, src_v, dst_v, a_v, rows, stage_v,
                  gsem, ssem, psem):
    cid = lax.axis_index("c")
    sid = lax.axis_index("s")

    pltpu.sync_copy(src_hbm.at[sid], src_v)
    pltpu.sync_copy(dst_hbm.at[sid], dst_v)
    for h in range(nh):
        _sc_head(h, cid, sid, ft_hbm, a12_hbm, outp, slab_hbm,
                 a12_v, amax_v, src_v, dst_v, a_v, rows, stage_v,
                 gsem, ssem, psem, do_elu)


class _SHNS:
    g_sh = None
    out_sh = None


_SH = _SHNS()


def _make_wrapped(nh, do_elu):
    def _sc_edge_wrapped(ft_hbm, a12_hbm, src_hbm, dst_hbm, outp, slab_hbm,
                         a12_v, amax_v, src_v, dst_v, a_v, rows, stage_v,
                         g_sh, out_sh, gsem, ssem, psem):
        _SH.g_sh, _SH.out_sh = g_sh, out_sh
        _sc_edge_body(nh, do_elu, ft_hbm, a12_hbm, src_hbm, dst_hbm, outp,
                      slab_hbm, a12_v, amax_v, src_v, dst_v, a_v, rows,
                      stage_v, gsem, ssem, psem)
    return _sc_edge_wrapped


def _sc_edge(ft, a12, src_rs, dst_rs, do_elu=False):
    nh = a12.shape[0]
    mesh = plsc.VectorSubcoreMesh(core_axis_name="c", subcore_axis_name="s")
    f = functools.partial(
        pl.kernel,
        out_type=[
            jax.ShapeDtypeStruct((NC, nh, NS, SEG, HID2), jnp.float32),
            jax.ShapeDtypeStruct((NC, NS, NPAD), jnp.float32),
        ],
        mesh=mesh,
        compiler_params=pltpu.CompilerParams(
            needs_layout_passes=False, use_tc_tiling_on_sc=False),
        scratch_types=[
            pltpu.VMEM((2 * NPAD,), jnp.float32),
            pltpu.VMEM((NPAD,), jnp.float32),
            pltpu.VMEM((NCHUNK, CH), jnp.int32),
            pltpu.VMEM((NCHUNK, CH), jnp.int32),
            pltpu.VMEM((NCHUNK, CH), jnp.float32),
            [pltpu.VMEM((CH, HID2), jnp.float32)] * K3,
            pltpu.VMEM((NS, SEG // 2), jnp.float32),
            pltpu.VMEM_SHARED((NPAD,), jnp.float32),
            pltpu.VMEM_SHARED((NPAD, HID2), jnp.float32),
            pltpu.SemaphoreType.DMA,
            pltpu.SemaphoreType.DMA,
            pltpu.SemaphoreType.DMA,
        ],
    )(_make_wrapped(nh, do_elu))
    outp, _unused = f(ft, a12, src_rs, dst_rs)
    return outp


def kernel(features, src, dst, Wh, bh, alw, alb, arw, arb,
           Wo, bo, alow, alob, arow, arob):
    src_rs = src.astype(jnp.int32).reshape(NS, NCHUNK, CH)
    dst_rs = dst.astype(jnp.int32).reshape(NS, NCHUNK, CH)

    bh3 = bh[:, None, :]
    aw2 = jnp.stack([alw, arw], axis=-1)
    ab2 = jnp.stack([alb, arb], axis=-1)[:, None, :]
    ft, a12 = _dense1(features, Wh, bh3, aw2, ab2)

    outp1 = _sc_edge(ft.reshape(HEADS * NC, NPAD, HID2),
                     a12.reshape(HEADS, 2 * NPAD), src_rs, dst_rs)
    p0 = outp1[0].reshape(HEADS, NPAD, HID2)
    p1 = outp1[1].reshape(HEADS, NPAD, HID2)

    Wo3 = Wo.reshape(HEADS, HID, NCLS)
    Wo2a = Wo3[:, :HID2, :]
    Wo2b = Wo3[:, HID2:, :]
    bo2 = bo[None, :]
    aw22 = jnp.stack([alow, arow], axis=-1)
    ab22 = jnp.stack([alob, arob])[None, :]
    _ft2_full, ft2h, a12_2 = _dense2(p0, p1, Wo2a, Wo2b, bo2, aw22, ab22)

    outp2 = _sc_edge(ft2h, a12_2.reshape(1, 2 * NPAD), src_rs, dst_rs,
                     do_elu=True)
    q0 = outp2[0].reshape(NPAD, HID2)
    q1 = outp2[1].reshape(NPAD, HID2)
    return jnp.concatenate([q0, q1], axis=1)[:N]


# speedup vs baseline: 43.5017x; 1.0141x over previous
import functools

import jax
import jax.numpy as jnp
from jax import lax
from jax.experimental import pallas as pl
from jax.experimental.pallas import tpu as pltpu
from jax.experimental.pallas import tpu_sc as plsc

N = 10000
E = 320000
IN_DIM = 128
HID = 64
HEADS = 4
NCLS = 64

NPAD = 10240
NC, NS, L = 2, 16, 16
HID2 = HID // 2
CH = 80
NCHUNK = (E // NS) // CH
SEG = NPAD // NS
VPC = CH // L
K2 = 5
K3 = 5


def _dense1_body(x_ref, w_ref, b_ref, aw_ref, ab_ref, ft_ref, a12_ref):
    ft = jnp.dot(x_ref[...], w_ref[0], preferred_element_type=jnp.float32)
    ft = ft + b_ref[0]
    ft_ref[0, 0, pl.ds(0, N)] = ft[:, :HID2]
    ft_ref[0, 1, pl.ds(0, N)] = ft[:, HID2:]
    a12 = jnp.dot(ft, aw_ref[0], preferred_element_type=jnp.float32)
    a12_ref[0, pl.ds(0, N)] = a12 + ab_ref[0]


def _dense1(x, Wh, bh3, aw2, ab2):
    return pl.pallas_call(
        _dense1_body,
        grid=(HEADS,),
        in_specs=[
            pl.BlockSpec((N, IN_DIM), lambda h: (0, 0)),
            pl.BlockSpec((1, IN_DIM, HID), lambda h: (h, 0, 0)),
            pl.BlockSpec((1, 1, HID), lambda h: (h, 0, 0)),
            pl.BlockSpec((1, HID, 2), lambda h: (h, 0, 0)),
            pl.BlockSpec((1, 1, 2), lambda h: (h, 0, 0)),
        ],
        out_specs=[
            pl.BlockSpec((1, NC, NPAD, HID2), lambda h: (h, 0, 0, 0)),
            pl.BlockSpec((1, NPAD, 2), lambda h: (h, 0, 0)),
        ],
        out_shape=[
            jax.ShapeDtypeStruct((HEADS, NC, NPAD, HID2), jnp.float32),
            jax.ShapeDtypeStruct((HEADS, NPAD, 2), jnp.float32),
        ],
    )(x, Wh, bh3, aw2, ab2)


def _dense2_body(p0_ref, p1_ref, wa_ref, wb_ref, b_ref, aw_ref, ab_ref,
                 ft2_ref, fth_ref, a12_ref):
    h = pl.program_id(0)
    s0 = p0_ref[0]
    s1 = p1_ref[0]
    h0 = jnp.where(s0 > 0, s0, jnp.exp(s0) - 1.0)
    h1 = jnp.where(s1 > 0, s1, jnp.exp(s1) - 1.0)
    part = (jnp.dot(h0, wa_ref[0], preferred_element_type=jnp.float32)
            + jnp.dot(h1, wb_ref[0], preferred_element_type=jnp.float32))

    @pl.when(h == 0)
    def _():
        ft2_ref[...] = part + b_ref[...]

    @pl.when(h > 0)
    def _():
        ft2_ref[...] = ft2_ref[...] + part

    @pl.when(h == HEADS - 1)
    def _():
        ft2 = ft2_ref[...]
        fth_ref[0] = ft2[:, :HID2]
        fth_ref[1] = ft2[:, HID2:]
        a12_ref[...] = (jnp.dot(ft2, aw_ref[...],
                                preferred_element_type=jnp.float32)
                        + ab_ref[...])


def _dense2(p0, p1, Wo2a, Wo2b, bo2, aw22, ab22):
    return pl.pallas_call(
        _dense2_body,
        grid=(HEADS,),
        in_specs=[
            pl.BlockSpec((1, NPAD, HID2), lambda h: (h, 0, 0)),
            pl.BlockSpec((1, NPAD, HID2), lambda h: (h, 0, 0)),
            pl.BlockSpec((1, HID2, NCLS), lambda h: (h, 0, 0)),
            pl.BlockSpec((1, HID2, NCLS), lambda h: (h, 0, 0)),
            pl.BlockSpec((1, NCLS), lambda h: (0, 0)),
            pl.BlockSpec((NCLS, 2), lambda h: (0, 0)),
            pl.BlockSpec((1, 2), lambda h: (0, 0)),
        ],
        out_specs=[
            pl.BlockSpec((NPAD, NCLS), lambda h: (0, 0)),
            pl.BlockSpec((NC, NPAD, HID2), lambda h: (0, 0, 0)),
            pl.BlockSpec((NPAD, 2), lambda h: (0, 0)),
        ],
        out_shape=[
            jax.ShapeDtypeStruct((NPAD, NCLS), jnp.float32),
            jax.ShapeDtypeStruct((NC, NPAD, HID2), jnp.float32),
            jax.ShapeDtypeStruct((NPAD, 2), jnp.float32),
        ],
    )(p0, p1, Wo2a, Wo2b, bo2, aw22, ab22)


def _sc_head(h, cid, sid, ft_hbm, a12_hbm, outp, slab_hbm,
             a12_v, amax_v, src_v, dst_v, a_v, rows, stage_v,
             gsem, ssem, psem, do_elu):
    one16 = jnp.ones((L,), jnp.int32)
    fzero = jnp.zeros((L,), jnp.float32)
    HSEG = SEG // 2

    a12d = pltpu.async_copy(a12_hbm.at[h], a12_v, psem)

    def _z1(i, _):
        amax_v[pl.ds(i * L, L)] = fzero
        return 0
    lax.fori_loop(0, NPAD // L, _z1, 0)

    def _z2(i, _):
        stage_v[0, pl.ds(i * L, L)] = fzero
        return 0
    lax.fori_loop(0, HSEG // L, _z2, 0)
    azd = [pltpu.async_copy(
               stage_v.at[0],
               _SH.asum_sh.at[pl.ds(sid * SEG + half * HSEG, HSEG)],
               ssem)
           for half in range(2)]

    def _z3(r, _):
        for c in range(HID2 // L):
            rows[0][r, pl.ds(c * L, L)] = fzero
        return 0
    lax.fori_loop(0, CH, _z3, 0)
    zd = [pltpu.async_copy(rows[0],
                           _SH.out_sh.at[pl.ds(sid * SEG + b * CH, CH), :],
                           gsem)
          for b in range(SEG // CH)]
    a12d.wait()

    def _p1(j, _):
        for i in range(VPC):
            sl = pl.ds(i * L, L)
            d_idx = dst_v[j, sl]
            s_idx = src_v[j, sl]
            x = (plsc.load_gather(a12_v, [d_idx * 2])
                 + plsc.load_gather(a12_v, [s_idx * 2 + one16]))
            a = jnp.where(x > 0, x, 0.01 * x)
            a_v[j, sl] = a
            old = plsc.load_gather(amax_v, [d_idx])
            plsc.store_scatter(amax_v, [d_idx], jnp.maximum(old, a))
            chk = plsc.load_gather(amax_v, [d_idx])
            pend = chk < a

            def _wcond(m):
                return jnp.any(m)

            def _wbody(m):
                o2 = plsc.load_gather(amax_v, [d_idx], mask=m)
                plsc.store_scatter(amax_v, [d_idx], jnp.maximum(o2, a),
                                   mask=m)
                c2 = plsc.load_gather(amax_v, [d_idx], mask=m)
                return (c2 < a) & m

            lax.while_loop(_wcond, _wbody, pend)
        return 0
    lax.fori_loop(0, NCHUNK, _p1, 0)

    pltpu.sync_copy(amax_v, slab_hbm.at[cid, sid])
    for d in azd:
        d.wait()
    plsc.subcore_barrier()
    for half in range(2):
        hs = pl.ds(sid * SEG + half * HSEG, HSEG)
        pltpu.sync_copy(slab_hbm.at[cid, :, hs], stage_v)

        def _red(i, _):
            sl = pl.ds(i * L, L)
            m = stage_v[0, sl]
            for r in range(1, NS):
                m = jnp.maximum(m, stage_v[r, sl])
            stage_v[0, sl] = m
            return 0
        lax.fori_loop(0, HSEG // L, _red, 0)
        pltpu.sync_copy(stage_v.at[0], _SH.g_sh.at[hs])
    plsc.subcore_barrier()
    pltpu.sync_copy(_SH.g_sh, amax_v)

    def _p2g(g, _):
        descs = []
        for q in range(K2):
            j = g * K2 + q
            for i in range(VPC):
                sl = pl.ds(i * L, L)
                am = plsc.load_gather(amax_v, [dst_v[j, sl]])
                a_v[j, sl] = jnp.exp(a_v[j, sl] - am)
            descs.append(pltpu.async_copy(
                a_v.at[j], _SH.asum_sh.at[src_v.at[j]], psem, add=True))
        for d in descs:
            d.wait()
        return 0
    lax.fori_loop(0, NCHUNK // K2, _p2g, 0)
    for d in zd:
        d.wait()
    plsc.subcore_barrier()
    pltpu.sync_copy(_SH.asum_sh, amax_v)

    def _inv(i, _):
        sl = pl.ds(i * L, L)
        amax_v[sl] = 1.0 / amax_v[sl]
        return 0
    lax.fori_loop(0, NPAD // L, _inv, 0)

    def _p3g(g, _):
        gd = []
        for q in range(K3):
            j = g * K3 + q
            gd.append(pltpu.async_copy(
                ft_hbm.at[2 * h + cid].at[dst_v.at[j]], rows[q], gsem))
        sd = []
        for q in range(K3):
            j = g * K3 + q
            for i in range(VPC):
                sl = pl.ds(i * L, L)
                s = plsc.load_gather(amax_v, [dst_v[j, sl]])
                a_v[j, sl] = a_v[j, sl] * s
            gd[q].wait()
            rq = rows[q]

            def _mul(gg, _, j=j, rq=rq):
                av = a_v[j, pl.ds(gg * L, L)]
                for r16 in range(L):
                    r = gg * L + r16
                    t = jnp.broadcast_to(av[r16], (L,))
                    for c in range(HID2 // L):
                        csl = pl.ds(c * L, L)
                        rq[r, csl] = rq[r, csl] * t
                return 0
            lax.fori_loop(0, CH // L, _mul, 0)
            sd.append(pltpu.async_copy(
                rq, _SH.out_sh.at[src_v.at[j]], ssem, add=True))
        for d in sd:
            d.wait()
        return 0
    lax.fori_loop(0, NCHUNK // K3, _p3g, 0)

    plsc.subcore_barrier()
    if not do_elu:
        pltpu.sync_copy(_SH.out_sh.at[pl.ds(sid * SEG, SEG), :],
                        outp.at[cid, h, sid])
    else:
        nb = SEG // CH
        ind = {}
        outd = {}
        for b in range(min(K3, nb)):
            ind[b] = pltpu.async_copy(
                _SH.out_sh.at[pl.ds(sid * SEG + b * CH, CH), :],
                rows[b], gsem)
        for b in range(nb):
            q = b % K3
            ind[b].wait()

            def _elu(r, _, q=q):
                for c in range(HID2 // L):
                    csl = pl.ds(c * L, L)
                    v = rows[q][r, csl]
                    rows[q][r, csl] = jnp.where(v > 0, v, jnp.exp(v) - 1.0)
                return 0
            lax.fori_loop(0, CH, _elu, 0)
            outd[b] = pltpu.async_copy(
                rows[q], outp.at[cid, h, sid].at[pl.ds(b * CH, CH), :],
                ssem)
            nxt = b + K3
            if nxt < nb:
                outd[b].wait()
                del outd[b]
                ind[nxt] = pltpu.async_copy(
                    _SH.out_sh.at[pl.ds(sid * SEG + nxt * CH, CH), :],
                    rows[q], gsem)
        for b in sorted(outd):
            outd[b].wait()


def _sc_edge_body(nh, do_elu, ft_hbm, a12_hbm, src_hbm, dst_hbm, outp,
                  slab_hbm, a12_v, amax_v, src_v, dst_v, a_v, rows, stage_v,
                  gsem, ssem, psem):
    cid = lax.axis_index("c")
    sid = lax.axis_index("s")

    pltpu.sync_copy(src_hbm.at[sid], src_v)
    pltpu.sync_copy(dst_hbm.at[sid], dst_v)
    for h in range(nh):
        _sc_head(h, cid, sid, ft_hbm, a12_hbm, outp, slab_hbm,
                 a12_v, amax_v, src_v, dst_v, a_v, rows, stage_v,
                 gsem, ssem, psem, do_elu)


class _SHNS:
    g_sh = None
    asum_sh = None
    out_sh = None


_SH = _SHNS()


def _make_wrapped(nh, do_elu):
    def _sc_edge_wrapped(ft_hbm, a12_hbm, src_hbm, dst_hbm, outp, slab_hbm,
                         a12_v, amax_v, src_v, dst_v, a_v, rows, stage_v,
                         g_sh, asum_sh, out_sh, gsem, ssem, psem):
        _SH.g_sh, _SH.asum_sh, _SH.out_sh = g_sh, asum_sh, out_sh
        _sc_edge_body(nh, do_elu, ft_hbm, a12_hbm, src_hbm, dst_hbm, outp,
                      slab_hbm, a12_v, amax_v, src_v, dst_v, a_v, rows,
                      stage_v, gsem, ssem, psem)
    return _sc_edge_wrapped


def _sc_edge(ft, a12, src_rs, dst_rs, do_elu=False):
    nh = a12.shape[0]
    mesh = plsc.VectorSubcoreMesh(core_axis_name="c", subcore_axis_name="s")
    f = functools.partial(
        pl.kernel,
        out_type=[
            jax.ShapeDtypeStruct((NC, nh, NS, SEG, HID2), jnp.float32),
            jax.ShapeDtypeStruct((NC, NS, NPAD), jnp.float32),
        ],
        mesh=mesh,
        compiler_params=pltpu.CompilerParams(
            needs_layout_passes=False, use_tc_tiling_on_sc=False),
        scratch_types=[
            pltpu.VMEM((2 * NPAD,), jnp.float32),
            pltpu.VMEM((NPAD,), jnp.float32),
            pltpu.VMEM((NCHUNK, CH), jnp.int32),
            pltpu.VMEM((NCHUNK, CH), jnp.int32),
            pltpu.VMEM((NCHUNK, CH), jnp.float32),
            [pltpu.VMEM((CH, HID2), jnp.float32)] * K3,
            pltpu.VMEM((NS, SEG // 2), jnp.float32),
            pltpu.VMEM_SHARED((NPAD,), jnp.float32),
            pltpu.VMEM_SHARED((NPAD,), jnp.float32),
            pltpu.VMEM_SHARED((NPAD, HID2), jnp.float32),
            pltpu.SemaphoreType.DMA,
            pltpu.SemaphoreType.DMA,
            pltpu.SemaphoreType.DMA,
        ],
    )(_make_wrapped(nh, do_elu))
    outp, _unused = f(ft, a12, src_rs, dst_rs)
    return outp


def kernel(features, src, dst, Wh, bh, alw, alb, arw, arb,
           Wo, bo, alow, alob, arow, arob):
    src_rs = src.astype(jnp.int32).reshape(NS, NCHUNK, CH)
    dst_rs = dst.astype(jnp.int32).reshape(NS, NCHUNK, CH)

    bh3 = bh[:, None, :]
    aw2 = jnp.stack([alw, arw], axis=-1)
    ab2 = jnp.stack([alb, arb], axis=-1)[:, None, :]
    ft, a12 = _dense1(features, Wh, bh3, aw2, ab2)

    outp1 = _sc_edge(ft.reshape(HEADS * NC, NPAD, HID2),
                     a12.reshape(HEADS, 2 * NPAD), src_rs, dst_rs)
    p0 = outp1[0].reshape(HEADS, NPAD, HID2)
    p1 = outp1[1].reshape(HEADS, NPAD, HID2)

    Wo3 = Wo.reshape(HEADS, HID, NCLS)
    Wo2a = Wo3[:, :HID2, :]
    Wo2b = Wo3[:, HID2:, :]
    bo2 = bo[None, :]
    aw22 = jnp.stack([alow, arow], axis=-1)
    ab22 = jnp.stack([alob, arob])[None, :]
    _ft2_full, ft2h, a12_2 = _dense2(p0, p1, Wo2a, Wo2b, bo2, aw22, ab22)

    outp2 = _sc_edge(ft2h, a12_2.reshape(1, 2 * NPAD), src_rs, dst_rs,
                     do_elu=True)
    q0 = outp2[0].reshape(NPAD, HID2)
    q1 = outp2[1].reshape(NPAD, HID2)
    return jnp.concatenate([q0, q1], axis=1)[:N]
